# baseline (speedup 1.0000x reference)
import math

import jax
import jax.numpy as jnp
from jax.experimental import pallas as pl

EPS = 1e-8


def _gmm_body(w_ref, pis_ref, mus_ref, sig_ref, t_ref, out_ref):
    w = w_ref[...]
    pis = jnp.abs(pis_ref[...])
    pi_norm = pis / jnp.sum(pis)
    sig = sig_ref[...]
    mus = mus_ref[...]
    sig2 = sig * sig
    log2e = 1.4426950408889634
    a = (-0.5 * log2e) / sig2
    b = -2.0 * a * mus
    c0 = a * mus * mus + (jnp.log(pi_norm) - 0.5 * jnp.log(2.0 * math.pi * sig2)) * log2e

    w2 = w * w
    le = a[:, :, None] * w2[None, :, :] + (b[:, :, None] * w[None, :, :] + c0[:, :, None])
    e = jnp.exp2(le)
    s = jnp.sum(e, axis=0)
    m = jnp.max(e, axis=0)
    c = log2e / (t_ref[0, 0] * (s + EPS))
    mc = m * c
    p = jnp.exp2(e * c[None, :, :] - mc[None, :, :])
    denom = jnp.sum(p, axis=0)
    num = jnp.sum(p * mus[:, :, None], axis=0)
    out_ref[...] = num / denom


def kernel(weights, mu, pi_k, pi_zero, sigma, sigma_zero, temperature):
    K = mu.shape[0] + 1
    R, C = weights.shape
    pis = jnp.concatenate([pi_zero, pi_k]).reshape(K, 1)
    mus = jnp.concatenate([jnp.zeros((1,), weights.dtype), mu]).reshape(K, 1)
    sigmas = jnp.concatenate([sigma_zero, sigma]).reshape(K, 1)
    temp = temperature.reshape(1, 1)

    B = 32
    grid = (R // B,)
    out = pl.pallas_call(
        _gmm_body,
        grid=grid,
        in_specs=[
            pl.BlockSpec((B, C), lambda i: (i, 0)),
            pl.BlockSpec((K, 1), lambda i: (0, 0)),
            pl.BlockSpec((K, 1), lambda i: (0, 0)),
            pl.BlockSpec((K, 1), lambda i: (0, 0)),
            pl.BlockSpec((1, 1), lambda i: (0, 0)),
        ],
        out_specs=pl.BlockSpec((B, C), lambda i: (i, 0)),
        out_shape=jax.ShapeDtypeStruct((R, C), weights.dtype),
    )(weights, pis, mus, sigmas, temp)
    return out


# speedup vs baseline: 1.3152x; 1.3152x over previous
import math

import jax
import jax.numpy as jnp
from jax.experimental import pallas as pl
from jax.experimental.pallas import tpu as pltpu

EPS = 1e-8


def _gmm_body(w_ref, pis_ref, mus_ref, sig_ref, t_ref, out_ref):
    w = w_ref[...]
    pis = jnp.abs(pis_ref[...])
    pi_norm = pis / jnp.sum(pis)
    sig = sig_ref[...]
    mus = mus_ref[...]
    sig2 = sig * sig
    log2e = 1.4426950408889634
    a = (-0.5 * log2e) / sig2
    b = -2.0 * a * mus
    c0 = a * mus * mus + (jnp.log(pi_norm) - 0.5 * jnp.log(2.0 * math.pi * sig2)) * log2e

    w2 = w * w
    le = a[:, :, None] * w2[None, :, :] + (b[:, :, None] * w[None, :, :] + c0[:, :, None])
    e = jnp.exp2(le)
    s = jnp.sum(e, axis=0)
    c = log2e / (t_ref[0, 0] * (s + EPS))
    p = jnp.exp2(e * c[None, :, :] - 128.0)
    denom = jnp.sum(p, axis=0)
    num = jnp.sum(p * mus[:, :, None], axis=0)
    out_ref[...] = num / denom


def kernel(weights, mu, pi_k, pi_zero, sigma, sigma_zero, temperature):
    K = mu.shape[0] + 1
    R, C = weights.shape
    pis = jnp.concatenate([pi_zero, pi_k]).reshape(K, 1)
    mus = jnp.concatenate([jnp.zeros((1,), weights.dtype), mu]).reshape(K, 1)
    sigmas = jnp.concatenate([sigma_zero, sigma]).reshape(K, 1)
    temp = temperature.reshape(1, 1)

    B = 16
    grid = (R // B,)
    out = pl.pallas_call(
        _gmm_body,
        grid=grid,
        in_specs=[
            pl.BlockSpec((B, C), lambda i: (i, 0)),
            pl.BlockSpec((K, 1), lambda i: (0, 0)),
            pl.BlockSpec((K, 1), lambda i: (0, 0)),
            pl.BlockSpec((K, 1), lambda i: (0, 0)),
            pl.BlockSpec((1, 1), lambda i: (0, 0)),
        ],
        out_specs=pl.BlockSpec((B, C), lambda i: (i, 0)),
        out_shape=jax.ShapeDtypeStruct((R, C), weights.dtype),
        compiler_params=pltpu.CompilerParams(
            dimension_semantics=("parallel",)),
    )(weights, pis, mus, sigmas, temp)
    return out
